# baseline (speedup 1.0000x reference)
import functools

import jax
import jax.numpy as jnp
from jax import lax
from jax.experimental import pallas as pl
from jax.experimental.pallas import tpu as pltpu
from jax.experimental.pallas import tpu_sc as plsc

N = 10000
NPAD = 10240
F = 256
ESL = 170000
EPAD = 172032
B = 128
BLKS = 42
ROWS_PER_TILE = NPAD // 16
NCHUNK = ROWS_PER_TILE // B

_mesh = plsc.VectorSubcoreMesh(
    core_axis_name="c", subcore_axis_name="s", num_cores=2, num_subcores=16)


def _tca_body(x_ref, w_ref, s_ref, d_ref, srcT_ref, dstT_ref):
    h = jnp.dot(x_ref[...], w_ref[...], preferred_element_type=jnp.float32)
    a16 = jnp.dot(h, s_ref[...], preferred_element_type=jnp.float32)
    srcT_ref[...] = jnp.concatenate([a16, h], axis=1)
    dstT_ref[...] = jnp.dot(h, d_ref[...], preferred_element_type=jnp.float32)


def _tca(x, W1cm, Scm, Dcm):
    return pl.pallas_call(
        _tca_body,
        grid=(NPAD // 1024,),
        in_specs=[
            pl.BlockSpec((1024, F), lambda i: (i, 0)),
            pl.BlockSpec((F, 64), lambda i: (0, 0)),
            pl.BlockSpec((64, 16), lambda i: (0, 0)),
            pl.BlockSpec((64, 16), lambda i: (0, 0)),
        ],
        out_specs=[
            pl.BlockSpec((1024, 80), lambda i: (i, 0)),
            pl.BlockSpec((1024, 16), lambda i: (i, 0)),
        ],
        out_shape=[
            jax.ShapeDtypeStruct((NPAD, 80), jnp.float32),
            jax.ShapeDtypeStruct((NPAD, 16), jnp.float32),
        ],
    )(x, W1cm, Scm, Dcm)


NPAIR = BLKS // 2


def _make_sc_body(SW, DW, CW, edge_fn):
    def body(srcT, dstT, srcI, dstI, out,
             idxs, idxd, rows_s, rows_d, contrib, acc,
             gsa, gda, sca, gsb, gdb, scb):
        c = lax.axis_index("c")
        s = lax.axis_index("s")
        tile = c * 16 + s
        gs = (gsa, gsb)
        gd = (gda, gdb)
        sc = (sca, scb)

        pltpu.sync_copy(srcI.at[pl.ds(tile * BLKS, BLKS)], idxs)
        pltpu.sync_copy(dstI.at[pl.ds(tile * BLKS, BLKS)], idxd)

        z16 = jnp.zeros((16,), jnp.float32)

        def zrow(i, _):
            for j in range(CW // 16):
                contrib[0, i, pl.ds(16 * j, 16)] = z16
            return 0
        lax.fori_loop(0, B, zrow, 0)

        def zcp(k, _):
            pltpu.sync_copy(contrib.at[0],
                            acc.at[pl.ds(s * ROWS_PER_TILE + k * B, B)])
            return 0
        lax.fori_loop(0, NCHUNK, zcp, 0)
        plsc.subcore_barrier()

        def issue(b, p):
            pltpu.async_copy(srcT.at[idxs.at[b]], rows_s.at[p], gs[p])
            pltpu.async_copy(dstT.at[idxd.at[b]], rows_d.at[p], gd[p])

        def wait_gather(b, p):
            pltpu.make_async_copy(srcT.at[idxs.at[b]], rows_s.at[p], gs[p]).wait()
            pltpu.make_async_copy(dstT.at[idxd.at[b]], rows_d.at[p], gd[p]).wait()

        def compute(p):
            @plsc.parallel_loop(0, B, unroll=4)
            def _(i):
                edge_fn(rows_s, rows_d, contrib, p, i)

        def scat(b, p):
            pltpu.async_copy(contrib.at[p], acc.at[idxd.at[b]], sc[p], add=True)

        def wait_scat(b, p):
            pltpu.make_async_copy(contrib.at[p], acc.at[idxd.at[b]], sc[p]).wait()

        issue(0, 0)
        issue(1, 1)
        wait_gather(0, 0)
        compute(0)
        scat(0, 0)
        issue(2, 0)
        wait_gather(1, 1)
        compute(1)
        scat(1, 1)
        issue(3, 1)

        def step(t, _):
            b0 = 2 * t
            b1 = b0 + 1
            wait_gather(b0, 0)
            wait_scat(b0 - 2, 0)
            compute(0)
            scat(b0, 0)
            issue(b0 + 2, 0)
            wait_gather(b1, 1)
            wait_scat(b1 - 2, 1)
            compute(1)
            scat(b1, 1)
            issue(b1 + 2, 1)
            return 0
        lax.fori_loop(1, NPAIR - 1, step, 0)

        bl = BLKS - 2
        wait_gather(bl, 0)
        wait_scat(bl - 2, 0)
        compute(0)
        scat(bl, 0)
        wait_gather(bl + 1, 1)
        wait_scat(bl - 1, 1)
        compute(1)
        scat(bl + 1, 1)
        wait_scat(bl, 0)
        wait_scat(bl + 1, 1)
        plsc.subcore_barrier()

        def drain(k, _):
            r0 = s * ROWS_PER_TILE + k * B
            pltpu.sync_copy(acc.at[pl.ds(r0, B)], contrib.at[0])
            pltpu.sync_copy(contrib.at[0], out.at[c, pl.ds(r0, B)])
            return 0
        lax.fori_loop(0, NCHUNK, drain, 0)

    return body


def _edge1(rows_s, rows_d, contrib, p, i):
    e = rows_s[p, i, pl.ds(0, 16)] + rows_d[p, i, pl.ds(0, 16)]
    w = jnp.exp(jnp.maximum(e, 0.2 * e))
    contrib[p, i, pl.ds(0, 16)] = w
    for q in range(4):
        contrib[p, i, pl.ds(16 + 16 * q, 16)] = (
            rows_s[p, i, pl.ds(16 + 16 * q, 16)] * w)


def _edge2(rows_s, rows_d, contrib, p, i):
    e = rows_s[p, i, pl.ds(16, 16)] + rows_d[p, i, pl.ds(0, 16)]
    w = jnp.exp(jnp.maximum(e, 0.2 * e))
    contrib[p, i, pl.ds(0, 16)] = rows_s[p, i, pl.ds(0, 16)] * w


def _sc_scratch(SW, DW, CW):
    return [
        pltpu.VMEM((BLKS, B), jnp.int32),
        pltpu.VMEM((BLKS, B), jnp.int32),
        pltpu.VMEM((2, B, SW), jnp.float32),
        pltpu.VMEM((2, B, DW), jnp.float32),
        pltpu.VMEM((2, B, CW), jnp.float32),
        pltpu.VMEM_SHARED((NPAD, CW), jnp.float32),
        pltpu.SemaphoreType.DMA,
        pltpu.SemaphoreType.DMA,
        pltpu.SemaphoreType.DMA,
        pltpu.SemaphoreType.DMA,
        pltpu.SemaphoreType.DMA,
        pltpu.SemaphoreType.DMA,
    ]


_sc_params = pltpu.CompilerParams(use_tc_tiling_on_sc=False)

_sc1 = functools.partial(
    pl.kernel,
    out_type=jax.ShapeDtypeStruct((2, NPAD, 80), jnp.float32),
    mesh=_mesh,
    compiler_params=_sc_params,
    scratch_types=_sc_scratch(80, 16, 80),
)(_make_sc_body(80, 16, 80, _edge1))


def _tcb_body(accs_ref, r_ref, b1_ref, w2_ref, as_ref, ad_ref,
              srcT_ref, dstT_ref):
    acc = accs_ref[0] + accs_ref[1]
    den = jnp.dot(acc[:, 0:8], r_ref[...],
                  preferred_element_type=jnp.float32) + 1e-16
    out1 = acc[:, 16:80] / den + b1_ref[...]
    g = jnp.where(out1 > 0, out1, jnp.exp(out1) - 1.0)
    h2 = jnp.dot(g, w2_ref[...], preferred_element_type=jnp.float32)
    asrc2 = jnp.sum(h2 * as_ref[...], axis=1, keepdims=True)
    adst2 = jnp.sum(h2 * ad_ref[...], axis=1, keepdims=True)
    srcT_ref[...] = jnp.concatenate(
        [jnp.ones((1024, 1), jnp.float32), h2,
         jnp.zeros((1024, 8), jnp.float32),
         jnp.broadcast_to(asrc2, (1024, 16))], axis=1)
    dstT_ref[...] = jnp.broadcast_to(adst2, (1024, 16))


def _tcb(accs, Rcm, b1cm, W2cm, a2s_row, a2d_row):
    return pl.pallas_call(
        _tcb_body,
        grid=(NPAD // 1024,),
        in_specs=[
            pl.BlockSpec((2, 1024, 80), lambda i: (0, i, 0)),
            pl.BlockSpec((8, 64), lambda i: (0, 0)),
            pl.BlockSpec((1, 64), lambda i: (0, 0)),
            pl.BlockSpec((64, 7), lambda i: (0, 0)),
            pl.BlockSpec((1, 7), lambda i: (0, 0)),
            pl.BlockSpec((1, 7), lambda i: (0, 0)),
        ],
        out_specs=[
            pl.BlockSpec((1024, 32), lambda i: (i, 0)),
            pl.BlockSpec((1024, 16), lambda i: (i, 0)),
        ],
        out_shape=[
            jax.ShapeDtypeStruct((NPAD, 32), jnp.float32),
            jax.ShapeDtypeStruct((NPAD, 16), jnp.float32),
        ],
    )(accs, Rcm, b1cm, W2cm, a2s_row, a2d_row)


_sc2 = functools.partial(
    pl.kernel,
    out_type=jax.ShapeDtypeStruct((2, NPAD, 16), jnp.float32),
    mesh=_mesh,
    compiler_params=_sc_params,
    scratch_types=_sc_scratch(32, 16, 16),
)(_make_sc_body(32, 16, 16, _edge2))


def _tcc_body(a_ref, b2_ref, out_ref, emb_ref):
    acc = a_ref[0] + a_ref[1]
    den = acc[:, 0:1] + 1e-16
    emb = acc[:, 1:8] / den + b2_ref[...]
    m = jnp.max(emb, axis=1, keepdims=True)
    p = jnp.exp(emb - m)
    out_ref[...] = p / jnp.sum(p, axis=1, keepdims=True)
    emb_ref[...] = emb


def _tcc(accs2, b2_row):
    return pl.pallas_call(
        _tcc_body,
        grid=(N // 1000,),
        in_specs=[
            pl.BlockSpec((2, 1000, 16), lambda i: (0, i, 0)),
            pl.BlockSpec((1, 7), lambda i: (0, 0)),
        ],
        out_specs=[
            pl.BlockSpec((1000, 7), lambda i: (i, 0)),
            pl.BlockSpec((1000, 7), lambda i: (i, 0)),
        ],
        out_shape=[
            jax.ShapeDtypeStruct((N, 7), jnp.float32),
            jax.ShapeDtypeStruct((N, 7), jnp.float32),
        ],
    )(accs2, b2_row)


def kernel(x, edge_index, W1, a1_src, a1_dst, b1, W2, a2_src, a2_dst, b2):
    f32 = jnp.float32
    perm = (jnp.arange(64) % 8) * 8 + jnp.arange(64) // 8
    W1cm = W1[:, perm]
    b1cm = b1[perm].reshape(1, 64).astype(f32)
    W2cm = W2[perm, :]
    M = jnp.tile(jnp.eye(8, dtype=f32), (8, 2))
    Scm = a1_src.reshape(8, 8).T.reshape(64, 1).astype(f32) * M
    Dcm = a1_dst.reshape(8, 8).T.reshape(64, 1).astype(f32) * M
    Rcm = jnp.tile(jnp.eye(8, dtype=f32), (1, 8))
    a2s_row = a2_src.reshape(1, 7).astype(f32)
    a2d_row = a2_dst.reshape(1, 7).astype(f32)
    b2_row = b2.reshape(1, 7).astype(f32)

    loops = jnp.arange(N, dtype=jnp.int32)
    def _blocks(e, pad):
        return pad.at[:ESL].set(
            jnp.concatenate([e.astype(jnp.int32), loops])
        ).reshape(EPAD // B, B)
    dpad = N + jnp.arange(EPAD, dtype=jnp.int32) % (NPAD - N)
    src = _blocks(edge_index[0], jnp.full((EPAD,), N, jnp.int32))
    dst = _blocks(edge_index[1], dpad)

    srcT1, dstT1 = _tca(x, W1cm, Scm, Dcm)
    accs1 = _sc1(srcT1, dstT1, src, dst)
    srcT2, dstT2 = _tcb(accs1, Rcm, b1cm, W2cm, a2s_row, a2d_row)
    accs2 = _sc2(srcT2, dstT2, src, dst)
    out, emb = _tcc(accs2, b2_row)
    return (out, emb)


# speedup vs baseline: 1.0750x; 1.0750x over previous
import functools

import jax
import jax.numpy as jnp
from jax import lax
from jax.experimental import pallas as pl
from jax.experimental.pallas import tpu as pltpu
from jax.experimental.pallas import tpu_sc as plsc

N = 10000
NPAD = 10240
F = 256
ESL = 170000
EPAD = 172032
B = 128
BLKS = 42
ROWS_PER_TILE = NPAD // 16
NCHUNK = ROWS_PER_TILE // B

_mesh = plsc.VectorSubcoreMesh(
    core_axis_name="c", subcore_axis_name="s", num_cores=2, num_subcores=16)


def _tca_body(x_ref, w_ref, s_ref, d_ref, srcT_ref, dstT_ref):
    h = jnp.dot(x_ref[...], w_ref[...], preferred_element_type=jnp.float32)
    a16 = jnp.dot(h, s_ref[...], preferred_element_type=jnp.float32)
    srcT_ref[...] = jnp.concatenate([a16, h], axis=1)
    dstT_ref[...] = jnp.dot(h, d_ref[...], preferred_element_type=jnp.float32)


def _tca(x, W1cm, Scm, Dcm):
    return pl.pallas_call(
        _tca_body,
        grid=(NPAD // 1024,),
        in_specs=[
            pl.BlockSpec((1024, F), lambda i: (i, 0)),
            pl.BlockSpec((F, 64), lambda i: (0, 0)),
            pl.BlockSpec((64, 16), lambda i: (0, 0)),
            pl.BlockSpec((64, 16), lambda i: (0, 0)),
        ],
        out_specs=[
            pl.BlockSpec((1024, 80), lambda i: (i, 0)),
            pl.BlockSpec((1024, 16), lambda i: (i, 0)),
        ],
        out_shape=[
            jax.ShapeDtypeStruct((NPAD, 80), jnp.float32),
            jax.ShapeDtypeStruct((NPAD, 16), jnp.float32),
        ],
    )(x, W1cm, Scm, Dcm)


NPAIR = BLKS // 2


def _make_sc_body(SW, DW, CW, edge_fn):
    def body(srcT, dstT, srcI, dstI, out,
             idxs, idxd, rows_s, rows_d, contrib, acc,
             gsa, gda, sca, gsb, gdb, scb):
        c = lax.axis_index("c")
        s = lax.axis_index("s")
        tile = c * 16 + s
        gs = (gsa, gsb)
        gd = (gda, gdb)
        sc = (sca, scb)

        pltpu.sync_copy(srcI.at[pl.ds(tile * BLKS, BLKS)], idxs)
        pltpu.sync_copy(dstI.at[pl.ds(tile * BLKS, BLKS)], idxd)

        z16 = jnp.zeros((16,), jnp.float32)

        def zrow(i, _):
            for j in range(CW // 16):
                contrib[0, i, pl.ds(16 * j, 16)] = z16
            return 0
        lax.fori_loop(0, B, zrow, 0)

        def zcp(k, _):
            pltpu.sync_copy(contrib.at[0],
                            acc.at[pl.ds(s * ROWS_PER_TILE + k * B, B)])
            return 0
        lax.fori_loop(0, NCHUNK, zcp, 0)
        plsc.subcore_barrier()

        def issue(b, p):
            pltpu.async_copy(srcT.at[idxs.at[b]], rows_s.at[p], gs[p])
            pltpu.async_copy(dstT.at[idxd.at[b]], rows_d.at[p], gd[p])

        def wait_gather(b, p):
            pltpu.make_async_copy(srcT.at[idxs.at[b]], rows_s.at[p], gs[p]).wait()
            pltpu.make_async_copy(dstT.at[idxd.at[b]], rows_d.at[p], gd[p]).wait()

        def compute(p):
            @plsc.parallel_loop(0, B, unroll=4)
            def _(i):
                edge_fn(rows_s, rows_d, contrib, p, i)

        def scat(b, p):
            pltpu.async_copy(contrib.at[p], acc.at[idxd.at[b]], sc[p], add=True)

        def wait_scat(b, p):
            pltpu.make_async_copy(contrib.at[p], acc.at[idxd.at[b]], sc[p]).wait()

        issue(0, 0)
        issue(1, 1)
        wait_gather(0, 0)
        compute(0)
        scat(0, 0)
        issue(2, 0)
        wait_gather(1, 1)
        compute(1)
        scat(1, 1)
        issue(3, 1)

        def step(t, _):
            b0 = 2 * t
            b1 = b0 + 1
            wait_gather(b0, 0)
            wait_scat(b0 - 2, 0)
            compute(0)
            scat(b0, 0)
            issue(b0 + 2, 0)
            wait_gather(b1, 1)
            wait_scat(b1 - 2, 1)
            compute(1)
            scat(b1, 1)
            issue(b1 + 2, 1)
            return 0
        lax.fori_loop(1, NPAIR - 1, step, 0)

        bl = BLKS - 2
        wait_gather(bl, 0)
        wait_scat(bl - 2, 0)
        compute(0)
        scat(bl, 0)
        wait_gather(bl + 1, 1)
        wait_scat(bl - 1, 1)
        compute(1)
        scat(bl + 1, 1)
        wait_scat(bl, 0)
        wait_scat(bl + 1, 1)
        plsc.subcore_barrier()

        def drain(k, _):
            r0 = s * ROWS_PER_TILE + k * B
            pltpu.sync_copy(acc.at[pl.ds(r0, B)], contrib.at[0])
            pltpu.sync_copy(contrib.at[0], out.at[c, pl.ds(r0, B)])
            return 0
        lax.fori_loop(0, NCHUNK, drain, 0)

    return body


def _edge1(rows_s, rows_d, contrib, p, i):
    e = rows_s[p, i, pl.ds(0, 16)] + rows_d[p, i, pl.ds(0, 16)]
    w = jnp.exp(jnp.maximum(e, 0.2 * e))
    contrib[p, i, pl.ds(0, 16)] = w
    for q in range(4):
        contrib[p, i, pl.ds(16 + 16 * q, 16)] = (
            rows_s[p, i, pl.ds(16 + 16 * q, 16)] * w)


def _edge2(rows_s, rows_d, contrib, p, i):
    e = rows_s[p, i, pl.ds(16, 16)] + rows_d[p, i, pl.ds(0, 16)]
    w = jnp.exp(jnp.maximum(e, 0.2 * e))
    contrib[p, i, pl.ds(0, 16)] = rows_s[p, i, pl.ds(0, 16)] * w


def _sc_scratch(SW, DW, CW):
    return [
        pltpu.VMEM((BLKS, B), jnp.int32),
        pltpu.VMEM((BLKS, B), jnp.int32),
        pltpu.VMEM((2, B, SW), jnp.float32),
        pltpu.VMEM((2, B, DW), jnp.float32),
        pltpu.VMEM((2, B, CW), jnp.float32),
        pltpu.VMEM_SHARED((NPAD, CW), jnp.float32),
        pltpu.SemaphoreType.DMA,
        pltpu.SemaphoreType.DMA,
        pltpu.SemaphoreType.DMA,
        pltpu.SemaphoreType.DMA,
        pltpu.SemaphoreType.DMA,
        pltpu.SemaphoreType.DMA,
    ]


_sc_params = pltpu.CompilerParams(use_tc_tiling_on_sc=False)

_sc1 = functools.partial(
    pl.kernel,
    out_type=jax.ShapeDtypeStruct((2, NPAD, 80), jnp.float32),
    mesh=_mesh,
    compiler_params=_sc_params,
    scratch_types=_sc_scratch(80, 16, 80),
)(_make_sc_body(80, 16, 80, _edge1))


def _tcb_body(accs_ref, r_ref, b1_ref, w2_ref, as_ref, ad_ref,
              srcT_ref, dstT_ref):
    acc = accs_ref[0] + accs_ref[1]
    den = jnp.dot(acc[:, 0:8], r_ref[...],
                  preferred_element_type=jnp.float32) + 1e-16
    out1 = acc[:, 16:80] / den + b1_ref[...]
    g = jnp.where(out1 > 0, out1, jnp.exp(out1) - 1.0)
    h2 = jnp.dot(g, w2_ref[...], preferred_element_type=jnp.float32)
    asrc2 = jnp.sum(h2 * as_ref[...], axis=1, keepdims=True)
    adst2 = jnp.sum(h2 * ad_ref[...], axis=1, keepdims=True)
    srcT_ref[...] = jnp.concatenate(
        [jnp.ones((1024, 1), jnp.float32), h2,
         jnp.zeros((1024, 8), jnp.float32),
         jnp.broadcast_to(asrc2, (1024, 16))], axis=1)
    dstT_ref[...] = jnp.broadcast_to(adst2, (1024, 16))


def _tcb(accs, Rcm, b1cm, W2cm, a2s_row, a2d_row):
    return pl.pallas_call(
        _tcb_body,
        grid=(NPAD // 1024,),
        in_specs=[
            pl.BlockSpec((2, 1024, 80), lambda i: (0, i, 0)),
            pl.BlockSpec((8, 64), lambda i: (0, 0)),
            pl.BlockSpec((1, 64), lambda i: (0, 0)),
            pl.BlockSpec((64, 7), lambda i: (0, 0)),
            pl.BlockSpec((1, 7), lambda i: (0, 0)),
            pl.BlockSpec((1, 7), lambda i: (0, 0)),
        ],
        out_specs=[
            pl.BlockSpec((1024, 32), lambda i: (i, 0)),
            pl.BlockSpec((1024, 16), lambda i: (i, 0)),
        ],
        out_shape=[
            jax.ShapeDtypeStruct((NPAD, 32), jnp.float32),
            jax.ShapeDtypeStruct((NPAD, 16), jnp.float32),
        ],
    )(accs, Rcm, b1cm, W2cm, a2s_row, a2d_row)


_sc2 = functools.partial(
    pl.kernel,
    out_type=jax.ShapeDtypeStruct((2, NPAD, 16), jnp.float32),
    mesh=_mesh,
    compiler_params=_sc_params,
    scratch_types=_sc_scratch(32, 16, 16),
)(_make_sc_body(32, 16, 16, _edge2))


def _tcc_body(a_ref, b2_ref, out_ref, emb_ref):
    acc = a_ref[0] + a_ref[1]
    den = acc[:, 0:1] + 1e-16
    emb = acc[:, 1:8] / den + b2_ref[...]
    m = jnp.max(emb, axis=1, keepdims=True)
    p = jnp.exp(emb - m)
    out_ref[...] = p / jnp.sum(p, axis=1, keepdims=True)
    emb_ref[...] = emb


def _tcc(accs2, b2_row):
    return pl.pallas_call(
        _tcc_body,
        grid=(N // 1000,),
        in_specs=[
            pl.BlockSpec((2, 1000, 16), lambda i: (0, i, 0)),
            pl.BlockSpec((1, 7), lambda i: (0, 0)),
        ],
        out_specs=[
            pl.BlockSpec((1000, 7), lambda i: (i, 0)),
            pl.BlockSpec((1000, 7), lambda i: (i, 0)),
        ],
        out_shape=[
            jax.ShapeDtypeStruct((N, 7), jnp.float32),
            jax.ShapeDtypeStruct((N, 7), jnp.float32),
        ],
    )(accs2, b2_row)


def kernel(x, edge_index, W1, a1_src, a1_dst, b1, W2, a2_src, a2_dst, b2):
    f32 = jnp.float32
    perm = (jnp.arange(64) % 8) * 8 + jnp.arange(64) // 8
    W1cm = W1[:, perm]
    b1cm = b1[perm].reshape(1, 64).astype(f32)
    W2cm = W2[perm, :]
    M = jnp.tile(jnp.eye(8, dtype=f32), (8, 2))
    Scm = a1_src.reshape(8, 8).T.reshape(64, 1).astype(f32) * M
    Dcm = a1_dst.reshape(8, 8).T.reshape(64, 1).astype(f32) * M
    Rcm = jnp.tile(jnp.eye(8, dtype=f32), (1, 8))
    a2s_row = a2_src.reshape(1, 7).astype(f32)
    a2d_row = a2_dst.reshape(1, 7).astype(f32)
    b2_row = b2.reshape(1, 7).astype(f32)

    loops = jnp.arange(N, dtype=jnp.int32)
    def _blocks(e, pad):
        b2d = pad.at[:ESL].set(
            jnp.concatenate([e.astype(jnp.int32), loops])
        ).reshape(BLKS, 32, B)
        return b2d.transpose(1, 0, 2).reshape(EPAD // B, B)
    dpad = N + jnp.arange(EPAD, dtype=jnp.int32) % (NPAD - N)
    src = _blocks(edge_index[0], jnp.full((EPAD,), N, jnp.int32))
    dst = _blocks(edge_index[1], dpad)

    srcT1, dstT1 = _tca(x, W1cm, Scm, Dcm)
    accs1 = _sc1(srcT1, dstT1, src, dst)
    srcT2, dstT2 = _tcb(accs1, Rcm, b1cm, W2cm, a2s_row, a2d_row)
    accs2 = _sc2(srcT2, dstT2, src, dst)
    out, emb = _tcc(accs2, b2_row)
    return (out, emb)
